# baseline (speedup 1.0000x reference)
import jax
import jax.numpy as jnp
from jax import lax
from jax.experimental import pallas as pl
from jax.experimental.pallas import tpu as pltpu
from jax.experimental.pallas import tpu_sc as plsc

N = 50000
D = 256
G = 512

BR = 8192
NTB = 4
R0 = NTB * BR

NSC = 2
NSUB = 16
CHUNK = 128
NFULL = (N - R0) // CHUNK
TAIL = N - R0 - NFULL * CHUNK
DHALF = D // NSC
SEG_PER_SUB = G // NSUB
CPS = -(-NFULL // NSUB)
NB = 5
NSLOT = -(-CPS // NB) * NB
PRE = 2


def _sc_body(x_hbm, batch_hbm, out_hbm, idx_v, rows_v, tidx_v, trows_v,
             zero_v, acc_sh, *sems):
    in_sems, add_sems, tail_sem = sems[:NB], sems[NB:2 * NB], sems[2 * NB]
    c = lax.axis_index("c")
    s = lax.axis_index("s")
    col0 = c * DHALF

    nmine = (NFULL - s + NSUB - 1) // NSUB

    def _start_in(i, b):
        base = R0 + (s + i * NSUB) * CHUNK
        pltpu.async_copy(batch_hbm.at[pl.ds(base, CHUNK)], idx_v.at[i],
                         in_sems[b])
        pltpu.async_copy(
            x_hbm.at[pl.ds(base, CHUNK), pl.ds(col0, DHALF)],
            rows_v.at[b], in_sems[b])

    def _wait_in(b):
        pltpu.make_async_copy(batch_hbm.at[pl.ds(0, CHUNK)], idx_v.at[0],
                              in_sems[b]).wait()
        pltpu.make_async_copy(
            x_hbm.at[pl.ds(0, CHUNK), pl.ds(col0, DHALF)],
            rows_v.at[b], in_sems[b]).wait()

    def _start_add(j, b):
        pltpu.async_copy(rows_v.at[b], acc_sh.at[idx_v.at[j]],
                         add_sems[b], add=True)

    def _wait_add(b):
        pltpu.make_async_copy(
            x_hbm.at[pl.ds(0, CHUNK), pl.ds(col0, DHALF)],
            rows_v.at[b], add_sems[b]).wait()

    for b in range(PRE):
        _start_in(b, b)

    tbase = R0 + NFULL * CHUNK

    @pl.when(s == NSUB - 1)
    def _():
        pltpu.async_copy(batch_hbm.at[pl.ds(tbase, TAIL)], tidx_v.at[0],
                         tail_sem)
        pltpu.async_copy(
            x_hbm.at[pl.ds(tbase, TAIL), pl.ds(col0, DHALF)], trows_v,
            tail_sem)

    zrow = jnp.zeros((16,), jnp.float32)

    def _zr(i, _):
        for j in range(DHALF // 16):
            zero_v[i, pl.ds(j * 16, 16)] = zrow
        return 0

    lax.fori_loop(0, SEG_PER_SUB, _zr, 0)
    pltpu.sync_copy(zero_v, acc_sh.at[pl.ds(s * SEG_PER_SUB, SEG_PER_SUB)])
    plsc.subcore_barrier()

    def _slots(g, _):
        for b in range(NB):
            i = g * NB + b

            @pl.when((g >= 1) & (i - NB < nmine))
            def _():
                _wait_add(b)

            @pl.when((i >= PRE) & (i < nmine))
            def _():
                _start_in(i, b)

            j = i - 1
            bb = (b - 1) % NB

            @pl.when((j >= 0) & (j < nmine))
            def _():
                _wait_in(bb)
                _start_add(j, bb)
        return 0

    lax.fori_loop(0, NSLOT // NB, _slots, 0)

    jlast = NSLOT - 1
    blast = jlast % NB

    @pl.when(jlast < nmine)
    def _():
        _wait_in(blast)
        _start_add(jlast, blast)

    for k in range(NB):
        j2 = NSLOT - NB + k

        @pl.when(j2 < nmine)
        def _():
            _wait_add(j2 % NB)

    @pl.when(s == NSUB - 1)
    def _():
        pltpu.make_async_copy(batch_hbm.at[pl.ds(tbase, TAIL)],
                              tidx_v.at[0], tail_sem).wait()
        pltpu.make_async_copy(
            x_hbm.at[pl.ds(tbase, TAIL), pl.ds(col0, DHALF)], trows_v,
            tail_sem).wait()
        pltpu.sync_copy(trows_v, acc_sh.at[tidx_v.at[0]], add=True)

    plsc.subcore_barrier()

    pltpu.sync_copy(
        acc_sh.at[pl.ds(s * SEG_PER_SUB, SEG_PER_SUB)],
        out_hbm.at[pl.ds(s * SEG_PER_SUB, SEG_PER_SUB), pl.ds(col0, DHALF)])


W = 128


def _tc_body(batch_ref, x_ref, o_ref):
    pid = pl.program_id(0)
    b = batch_ref[...]
    xb = x_ref[...].astype(jnp.bfloat16)
    base = (jnp.minimum(batch_ref[0], G - W) // 8) * 8
    span_ok = batch_ref[BR - 1] - base < W

    @pl.when(pid == 0)
    def _():
        o_ref[...] = jnp.zeros_like(o_ref)

    @pl.when(span_ok)
    def _():
        oh = (lax.broadcasted_iota(jnp.int32, (W, BR), 0) + base
              == b[None, :]).astype(jnp.bfloat16)
        acc = jnp.dot(oh, xb, preferred_element_type=jnp.float32)
        o_ref[pl.ds(base, W), :] += acc

    @pl.when(jnp.logical_not(span_ok))
    def _():
        oh = (lax.broadcasted_iota(jnp.int32, (G, BR), 0)
              == b[None, :]).astype(jnp.bfloat16)
        o_ref[...] += jnp.dot(oh, xb, preferred_element_type=jnp.float32)


def _add_body(a_ref, b_ref, o_ref):
    o_ref[...] = a_ref[...] + b_ref[...]


def kernel(X, batch, num_graphs):
    del num_graphs
    batch = batch.astype(jnp.int32)

    mesh = plsc.VectorSubcoreMesh(core_axis_name="c", subcore_axis_name="s")
    sc_call = pl.kernel(
        _sc_body,
        out_type=jax.ShapeDtypeStruct((G, D), jnp.float32),
        mesh=mesh,
        scratch_types=[
            pltpu.VMEM((CPS, CHUNK), jnp.int32),
            pltpu.VMEM((NB, CHUNK, DHALF), jnp.float32),
            pltpu.VMEM((1, TAIL), jnp.int32),
            pltpu.VMEM((TAIL, DHALF), jnp.float32),
            pltpu.VMEM((SEG_PER_SUB, DHALF), jnp.float32),
            pltpu.VMEM_SHARED((G, DHALF), jnp.float32),
        ] + [pltpu.SemaphoreType.DMA] * (2 * NB + 1),
    )
    sc_part = sc_call(X, batch)

    tc_call = pl.pallas_call(
        _tc_body,
        grid=(NTB,),
        in_specs=[
            pl.BlockSpec((BR,), lambda i: (i,)),
            pl.BlockSpec((BR, D), lambda i: (i, 0)),
        ],
        out_specs=pl.BlockSpec((G, D), lambda i: (0, 0)),
        out_shape=jax.ShapeDtypeStruct((G, D), jnp.float32),
    )
    tc_part = tc_call(batch, X)

    add_call = pl.pallas_call(
        _add_body,
        out_shape=jax.ShapeDtypeStruct((G, D), jnp.float32),
    )
    return add_call(sc_part, tc_part)


if __name__ == "__main__":
    x = jnp.ones((N, D), jnp.float32)
    b = jnp.zeros((N,), jnp.int32)
    print(jax.jit(kernel)(x, b, G).shape)


# speedup vs baseline: 1.0040x; 1.0040x over previous
import jax
import jax.numpy as jnp
from jax import lax
from jax.experimental import pallas as pl
from jax.experimental.pallas import tpu as pltpu
from jax.experimental.pallas import tpu_sc as plsc

N = 50000
D = 256
G = 512

BR = 4096
NTB = 8
R0 = NTB * BR

NSC = 2
NSUB = 16
CHUNK = 128
NFULL = (N - R0) // CHUNK
TAIL = N - R0 - NFULL * CHUNK
DHALF = D // NSC
SEG_PER_SUB = G // NSUB
CPS = -(-NFULL // NSUB)
NB = 5
NSLOT = -(-CPS // NB) * NB
PRE = 2


def _sc_body(x_hbm, batch_hbm, out_hbm, idx_v, rows_v, tidx_v, trows_v,
             zero_v, acc_sh, *sems):
    in_sems, add_sems, tail_sem = sems[:NB], sems[NB:2 * NB], sems[2 * NB]
    c = lax.axis_index("c")
    s = lax.axis_index("s")
    col0 = c * DHALF

    nmine = (NFULL - s + NSUB - 1) // NSUB

    def _start_in(i, b):
        base = R0 + (s + i * NSUB) * CHUNK
        pltpu.async_copy(batch_hbm.at[pl.ds(base, CHUNK)], idx_v.at[i],
                         in_sems[b])
        pltpu.async_copy(
            x_hbm.at[pl.ds(base, CHUNK), pl.ds(col0, DHALF)],
            rows_v.at[b], in_sems[b])

    def _wait_in(b):
        pltpu.make_async_copy(batch_hbm.at[pl.ds(0, CHUNK)], idx_v.at[0],
                              in_sems[b]).wait()
        pltpu.make_async_copy(
            x_hbm.at[pl.ds(0, CHUNK), pl.ds(col0, DHALF)],
            rows_v.at[b], in_sems[b]).wait()

    def _start_add(j, b):
        pltpu.async_copy(rows_v.at[b], acc_sh.at[idx_v.at[j]],
                         add_sems[b], add=True)

    def _wait_add(b):
        pltpu.make_async_copy(
            x_hbm.at[pl.ds(0, CHUNK), pl.ds(col0, DHALF)],
            rows_v.at[b], add_sems[b]).wait()

    for b in range(PRE):
        _start_in(b, b)

    tbase = R0 + NFULL * CHUNK

    @pl.when(s == NSUB - 1)
    def _():
        pltpu.async_copy(batch_hbm.at[pl.ds(tbase, TAIL)], tidx_v.at[0],
                         tail_sem)
        pltpu.async_copy(
            x_hbm.at[pl.ds(tbase, TAIL), pl.ds(col0, DHALF)], trows_v,
            tail_sem)

    zrow = jnp.zeros((16,), jnp.float32)

    def _zr(i, _):
        for j in range(DHALF // 16):
            zero_v[i, pl.ds(j * 16, 16)] = zrow
        return 0

    lax.fori_loop(0, SEG_PER_SUB, _zr, 0)
    pltpu.sync_copy(zero_v, acc_sh.at[pl.ds(s * SEG_PER_SUB, SEG_PER_SUB)])
    plsc.subcore_barrier()

    def _slots(g, _):
        for b in range(NB):
            i = g * NB + b

            @pl.when((g >= 1) & (i - NB < nmine))
            def _():
                _wait_add(b)

            @pl.when((i >= PRE) & (i < nmine))
            def _():
                _start_in(i, b)

            j = i - 1
            bb = (b - 1) % NB

            @pl.when((j >= 0) & (j < nmine))
            def _():
                _wait_in(bb)
                _start_add(j, bb)
        return 0

    lax.fori_loop(0, NSLOT // NB, _slots, 0)

    jlast = NSLOT - 1
    blast = jlast % NB

    @pl.when(jlast < nmine)
    def _():
        _wait_in(blast)
        _start_add(jlast, blast)

    for k in range(NB):
        j2 = NSLOT - NB + k

        @pl.when(j2 < nmine)
        def _():
            _wait_add(j2 % NB)

    @pl.when(s == NSUB - 1)
    def _():
        pltpu.make_async_copy(batch_hbm.at[pl.ds(tbase, TAIL)],
                              tidx_v.at[0], tail_sem).wait()
        pltpu.make_async_copy(
            x_hbm.at[pl.ds(tbase, TAIL), pl.ds(col0, DHALF)], trows_v,
            tail_sem).wait()
        pltpu.sync_copy(trows_v, acc_sh.at[tidx_v.at[0]], add=True)

    plsc.subcore_barrier()

    pltpu.sync_copy(
        acc_sh.at[pl.ds(s * SEG_PER_SUB, SEG_PER_SUB)],
        out_hbm.at[pl.ds(s * SEG_PER_SUB, SEG_PER_SUB), pl.ds(col0, DHALF)])


W = 128


def _tc_body(batch_ref, x_ref, o_ref):
    pid = pl.program_id(0)
    b = batch_ref[...]
    xb = x_ref[...].astype(jnp.bfloat16)
    base = (jnp.minimum(batch_ref[0], G - W) // 8) * 8
    span_ok = batch_ref[BR - 1] - base < W

    @pl.when(pid == 0)
    def _():
        o_ref[...] = jnp.zeros_like(o_ref)

    @pl.when(span_ok)
    def _():
        oh = (lax.broadcasted_iota(jnp.int32, (W, BR), 0) + base
              == b[None, :]).astype(jnp.bfloat16)
        acc = jnp.dot(oh, xb, preferred_element_type=jnp.float32)
        o_ref[pl.ds(base, W), :] += acc

    @pl.when(jnp.logical_not(span_ok))
    def _():
        oh = (lax.broadcasted_iota(jnp.int32, (G, BR), 0)
              == b[None, :]).astype(jnp.bfloat16)
        o_ref[...] += jnp.dot(oh, xb, preferred_element_type=jnp.float32)


def _add_body(a_ref, b_ref, o_ref):
    o_ref[...] = a_ref[...] + b_ref[...]


def kernel(X, batch, num_graphs):
    del num_graphs
    batch = batch.astype(jnp.int32)

    mesh = plsc.VectorSubcoreMesh(core_axis_name="c", subcore_axis_name="s")
    sc_call = pl.kernel(
        _sc_body,
        out_type=jax.ShapeDtypeStruct((G, D), jnp.float32),
        mesh=mesh,
        scratch_types=[
            pltpu.VMEM((CPS, CHUNK), jnp.int32),
            pltpu.VMEM((NB, CHUNK, DHALF), jnp.float32),
            pltpu.VMEM((1, TAIL), jnp.int32),
            pltpu.VMEM((TAIL, DHALF), jnp.float32),
            pltpu.VMEM((SEG_PER_SUB, DHALF), jnp.float32),
            pltpu.VMEM_SHARED((G, DHALF), jnp.float32),
        ] + [pltpu.SemaphoreType.DMA] * (2 * NB + 1),
    )
    sc_part = sc_call(X, batch)

    tc_call = pl.pallas_call(
        _tc_body,
        grid=(NTB,),
        in_specs=[
            pl.BlockSpec((BR,), lambda i: (i,)),
            pl.BlockSpec((BR, D), lambda i: (i, 0)),
        ],
        out_specs=pl.BlockSpec((G, D), lambda i: (0, 0)),
        out_shape=jax.ShapeDtypeStruct((G, D), jnp.float32),
    )
    tc_part = tc_call(batch, X)

    add_call = pl.pallas_call(
        _add_body,
        out_shape=jax.ShapeDtypeStruct((G, D), jnp.float32),
    )
    return add_call(sc_part, tc_part)


if __name__ == "__main__":
    x = jnp.ones((N, D), jnp.float32)
    b = jnp.zeros((N,), jnp.int32)
    print(jax.jit(kernel)(x, b, G).shape)
